# baseline (speedup 1.0000x reference)
import dataclasses
import functools
import jax
import jax.numpy as jnp
from jax import lax
from jax.experimental import pallas as pl
from jax.experimental.pallas import tpu as pltpu
from jax.experimental.pallas import tpu_sc as plsc

_K = 5
_DIM = 3
_PATCH = 0.25
_KNN = 3

_BQ = 512
_NCP = 2560
_PADVAL = 1e4

_NF = 10000
_NP2 = 10240
_NPAD = 10240
_EPT = 5120
_E_PAD = 32 * _EPT
_CHUNK = 16
_NROW = _NP2 // 4


def _sc_compiler_params():
    cp = pltpu.CompilerParams()
    if "needs_layout_passes" in pltpu.CompilerParams.__dataclass_fields__:
        cp = dataclasses.replace(cp, needs_layout_passes=False)
    return cp


def _knn_body(q_ref, pT_ref, x_ref, o_ref):
    q = q_ref[...]
    pT = pT_ref[...]
    x = x_ref[...]
    qp = jnp.dot(q, pT, preferred_element_type=jnp.float32)
    pn = jnp.sum(pT * pT, axis=0, keepdims=True)
    qn = jnp.sum(q * q, axis=1, keepdims=True)
    d2 = qn + pn - 2.0 * qp
    colid = jax.lax.broadcasted_iota(jnp.int32, d2.shape, 1)
    mask = jnp.zeros(d2.shape, jnp.bool_)
    cur = d2
    for _ in range(_KNN):
        m = jnp.min(cur, axis=1, keepdims=True)
        is_min = cur == m
        first = jnp.min(jnp.where(is_min, colid, jnp.int32(2**30)),
                        axis=1, keepdims=True)
        sel = colid == first
        mask = mask | sel
        cur = jnp.where(sel, jnp.float32(1e30), cur)
    w = jnp.where(mask, 1.0 / jnp.maximum(d2, 1e-16), 0.0)
    s = jnp.sum(w, axis=1, keepdims=True)
    xi = jnp.dot(w, x, preferred_element_type=jnp.float32) / s
    o_ref[...] = xi


def _knn_interpolate_pallas(x, pos, pos_skip):
    Nc = pos.shape[0]
    Nf = pos_skip.shape[0]
    F = x.shape[1]
    nq_pad = ((Nf + _BQ - 1) // _BQ) * _BQ
    pT = jnp.full((8, _NCP), _PADVAL, jnp.float32)
    pT = pT.at[:3, :Nc].set(pos.T)
    pT = pT.at[3:, :Nc].set(0.0)
    xp = jnp.zeros((_NCP, F), jnp.float32).at[:Nc].set(x)
    qp = jnp.zeros((nq_pad, 8), jnp.float32).at[:Nf, :3].set(pos_skip)
    out = pl.pallas_call(
        _knn_body,
        grid=(nq_pad // _BQ,),
        in_specs=[
            pl.BlockSpec((_BQ, 8), lambda i: (i, 0)),
            pl.BlockSpec((8, _NCP), lambda i: (0, 0)),
            pl.BlockSpec((_NCP, F), lambda i: (0, 0)),
        ],
        out_specs=pl.BlockSpec((_BQ, F), lambda i: (i, 0)),
        out_shape=jax.ShapeDtypeStruct((nq_pad, F), jnp.float32),
    )(qp, pT, xp)
    return out[:Nf]


def _mm_body(h_ref, w_ref, o_ref):
    o_ref[...] = jnp.dot(h_ref[...], w_ref[...],
                         preferred_element_type=jnp.float32)


def _table_matmul(h, Wr):
    ci = h.shape[1]
    M = Wr.shape[1]
    return pl.pallas_call(
        _mm_body,
        grid=(_NPAD // _BQ,),
        in_specs=[
            pl.BlockSpec((_BQ, ci), lambda i: (i, 0)),
            pl.BlockSpec((ci, M), lambda i: (0, 0)),
        ],
        out_specs=pl.BlockSpec((_BQ, M), lambda i: (i, 0)),
        out_shape=jax.ShapeDtypeStruct((_NPAD, M), jnp.float32),
    )(h, Wr)


def _sc_layer(table, srcv_h, dstv_h, px, py, pz):
    mesh = plsc.VectorSubcoreMesh(core_axis_name="c", subcore_axis_name="s")

    @functools.partial(
        pl.kernel, mesh=mesh, compiler_params=_sc_compiler_params(),
        out_type=jax.ShapeDtypeStruct((2 * _NROW, 2, 128), jnp.float32),
        scratch_types=[
            pltpu.VMEM((_EPT,), jnp.int32),
            pltpu.VMEM((_EPT,), jnp.int32),
            pltpu.VMEM((_NP2,), jnp.float32),
            pltpu.VMEM((_NP2,), jnp.float32),
            pltpu.VMEM((_NP2,), jnp.float32),
            pltpu.VMEM((128,), jnp.int32),
            pltpu.VMEM((128,), jnp.int32),
            pltpu.VMEM((128, 128), jnp.float32),
            pltpu.VMEM((128, 128), jnp.float32),
            pltpu.VMEM((16, 2, 128), jnp.float32),
            pltpu.VMEM((16, 2, 128), jnp.float32),
            pltpu.VMEM((1, 16), jnp.int32),
            pltpu.VMEM((1, 16), jnp.int32),
            pltpu.SemaphoreType.DMA,
            pltpu.SemaphoreType.DMA,
            pltpu.SemaphoreType.DMA,
            pltpu.SemaphoreType.DMA,
            pltpu.VMEM_SHARED((_NROW, 2, 128), jnp.float32),
        ],
    )
    def k(table_h, src_h, dst_h, px_h, py_h, pz_h, out_h,
          srcv, dstv, pxv, pyv, pzv, qidxA, qidxB, gbufA, gbufB,
          sbufA, sbufB, dstrowA, dstrowB, semgA, semgB, semsA, semsB,
          agg_sh):
        cid = lax.axis_index("c")
        sid = lax.axis_index("s")
        wid = sid * 2 + cid

        zero16f = jnp.zeros((16,), jnp.float32)
        zero16i = jnp.zeros((16,), jnp.int32)

        @pl.loop(0, 16)
        def _(r):
            for t in range(2):
                for j in range(8):
                    sbufA.at[r, t][pl.ds(j * 16, 16)] = zero16f
                    sbufB.at[r, t][pl.ds(j * 16, 16)] = zero16f

        zbase = sid * (_NROW // 16)

        @pl.loop(0, _NROW // 16 // 16)
        def _(part):
            pltpu.sync_copy(sbufA, agg_sh.at[pl.ds(zbase + part * 16, 16)])
        plsc.subcore_barrier()

        base = wid * _EPT
        pltpu.sync_copy(src_h.at[pl.ds(base, _EPT)], srcv)
        pltpu.sync_copy(dst_h.at[pl.ds(base, _EPT)], dstv)
        pltpu.sync_copy(px_h, pxv)
        pltpu.sync_copy(py_h, pyv)
        pltpu.sync_copy(pz_h, pzv)

        def phase_p(c, qidx_q, gbuf_q, semg):
            off = c * _CHUNK
            s = srcv[pl.ds(off, 16)]
            d = dstv[pl.ds(off, 16)]

            def dim_corners(pv):
                gs = plsc.load_gather(pv, [s])
                gd = plsc.load_gather(pv, [d])
                v = jnp.clip((gd - gs) * 16.0 + 2.0, 0.0, 4.0)
                loi = v.astype(jnp.int32)
                fr = v - loi.astype(jnp.float32)
                return fr, loi, jnp.minimum(loi + 1, 4)

            fx, kx0, kx1 = dim_corners(pxv)
            fy, ky0, ky1 = dim_corners(pyv)
            fz, kz0, kz1 = dim_corners(pzv)
            gb = s * 125
            ax0 = gb + kx0
            ax1 = gb + kx1
            by0 = ky0 * 5
            by1 = ky1 * 5
            bz0 = kz0 * 25
            bz1 = kz1 * 25
            fx0 = 1.0 - fx
            fy0 = 1.0 - fy
            fz0 = 1.0 - fz
            wxy = (fx0 * fy0, fx0 * fy, fx * fy0, fx * fy)
            axy = (ax0 + by0, ax0 + by1, ax1 + by0, ax1 + by1)

            ws = []
            ps = []
            g = 0
            for t in range(4):
                for (fz_t, bz_t) in ((fz0, bz0), (fz, bz1)):
                    w8 = wxy[t] * fz_t
                    g8 = axy[t] + bz_t
                    qidx_q[pl.ds(g * 16, 16)] = g8 >> 1
                    ws.append(w8)
                    ps.append((g8 & 1) << 6)
                    g += 1

            pltpu.async_copy(table_h.at[qidx_q], gbuf_q, semg)
            return tuple(ws) + tuple(ps) + (d,)

        def phase_d(st, qidx_q, gbuf_q, sbuf_q, dstrow_q, semg, sems):
            ws = st[0:8]
            ps = st[8:16]
            d = st[16]
            pltpu.make_async_copy(table_h.at[qidx_q], gbuf_q, semg).wait()
            pltpu.make_async_copy(sbuf_q, agg_sh.at[dstrow_q.at[0]],
                                  sems).wait()
            dstrow_q.at[0][pl.ds(0, 16)] = d >> 2
            qv = (d & 3) << 6

            for e in range(16):
                accs = None
                for g in range(8):
                    w = ws[g][e]
                    p = ps[g][e]
                    row = gbuf_q.at[g * 16 + e]
                    vals = [row[pl.ds(p + j * 16, 16)] * w for j in range(4)]
                    if accs is None:
                        accs = vals
                    else:
                        accs = [a + v for a, v in zip(accs, vals)]
                qo = qv[e]
                for q in range(4):
                    t_s = q // 2
                    o_s = (q % 2) * 64
                    m = qo == q * 64
                    for j in range(4):
                        sbuf_q.at[e, t_s][pl.ds(o_s + j * 16, 16)] = (
                            jnp.where(m, accs[j], zero16f))

            pltpu.async_copy(sbuf_q, agg_sh.at[dstrow_q.at[0]], sems,
                             add=True)

        dstrowA.at[0][pl.ds(0, 16)] = zero16i
        dstrowB.at[0][pl.ds(0, 16)] = zero16i
        pltpu.async_copy(sbufA, agg_sh.at[dstrowA.at[0]], semsA, add=True)
        pltpu.async_copy(sbufB, agg_sh.at[dstrowB.at[0]], semsB, add=True)

        stA0 = phase_p(0, qidxA, gbufA, semgA)

        def pair_body(i, stA):
            c = 2 * i
            stB = phase_p(c + 1, qidxB, gbufB, semgB)
            phase_d(stA, qidxA, gbufA, sbufA, dstrowA, semgA, semsA)
            stA2 = phase_p(c + 2, qidxA, gbufA, semgA)
            phase_d(stB, qidxB, gbufB, sbufB, dstrowB, semgB, semsB)
            return stA2

        nchunk = _EPT // _CHUNK
        stA = lax.fori_loop(0, nchunk // 2 - 1, pair_body, stA0)

        stB = phase_p(nchunk - 1, qidxB, gbufB, semgB)
        phase_d(stA, qidxA, gbufA, sbufA, dstrowA, semgA, semsA)
        phase_d(stB, qidxB, gbufB, sbufB, dstrowB, semgB, semsB)

        pltpu.make_async_copy(sbufA, agg_sh.at[dstrowA.at[0]], semsA).wait()
        pltpu.make_async_copy(sbufB, agg_sh.at[dstrowB.at[0]], semsB).wait()

        plsc.subcore_barrier()

        obase = sid * (_NROW // 16)

        @pl.loop(0, _NROW // 16 // 16)
        def _(part):
            ooff = obase + part * 16
            pltpu.sync_copy(agg_sh.at[pl.ds(ooff, 16)], sbufA)
            pltpu.sync_copy(sbufA, out_h.at[pl.ds(cid * _NROW + ooff, 16)])

    return k(table, srcv_h, dstv_h, px, py, pz)


def _sc_cnt(dstv_h):
    mesh = plsc.VectorSubcoreMesh(core_axis_name="c", subcore_axis_name="s")

    @functools.partial(
        pl.kernel, mesh=mesh, compiler_params=_sc_compiler_params(),
        out_type=jax.ShapeDtypeStruct((32 * _NP2,), jnp.float32),
        scratch_types=[
            pltpu.VMEM((_EPT,), jnp.int32),
            pltpu.VMEM((_NP2,), jnp.float32),
        ],
    )
    def k(dst_h, out_h, dstv, hist):
        cid = lax.axis_index("c")
        sid = lax.axis_index("s")
        wid = sid * 2 + cid
        zero16 = jnp.zeros((16,), jnp.float32)
        one16 = jnp.full((16,), 1.0, jnp.float32)

        @pl.loop(0, _NP2, step=16)
        def _(i):
            hist[pl.ds(i, 16)] = zero16

        base = wid * _EPT
        pltpu.sync_copy(dst_h.at[pl.ds(base, _EPT)], dstv)

        @pl.loop(0, _EPT // 16)
        def _(c):
            d = dstv[pl.ds(c * 16, 16)]
            plsc.addupdate_scatter(hist, [d], one16)

        pltpu.sync_copy(hist, out_h.at[pl.ds(wid * _NP2, _NP2)])

    return k(dstv_h)


def _fin_body(a0_ref, a1_ref, c_ref, h_ref, root_ref, b_ref, o_ref):
    agg = a0_ref[...] + a1_ref[...]
    cnt = jnp.maximum(c_ref[:, 0:1], 1.0)
    hr = jnp.dot(h_ref[...], root_ref[...], preferred_element_type=jnp.float32)
    out = agg / cnt + hr + b_ref[0:1, :]
    o_ref[...] = jnp.where(out > 0, out, jnp.exp(jnp.minimum(out, 0.0)) - 1.0)


def _finalize(agg0, agg1, cnt8, h, root, bias):
    ci = h.shape[1]
    bias8 = jnp.broadcast_to(bias[None, :], (8, 64))
    return pl.pallas_call(
        _fin_body,
        grid=(_NPAD // _BQ,),
        in_specs=[
            pl.BlockSpec((_BQ, 64), lambda i: (i, 0)),
            pl.BlockSpec((_BQ, 64), lambda i: (i, 0)),
            pl.BlockSpec((_BQ, 8), lambda i: (i, 0)),
            pl.BlockSpec((_BQ, ci), lambda i: (i, 0)),
            pl.BlockSpec((ci, 64), lambda i: (0, 0)),
            pl.BlockSpec((8, 64), lambda i: (0, 0)),
        ],
        out_specs=pl.BlockSpec((_BQ, 64), lambda i: (i, 0)),
        out_shape=jax.ShapeDtypeStruct((_NPAD, 64), jnp.float32),
    )(agg0, agg1, cnt8, h, root, bias8)


def kernel(x, pos, batch, x_skip, pos_skip, batch_skip, edge_index,
           W0, root0, b0, W1, root1, b1, W2, root2, b2):
    xi = _knn_interpolate_pallas(x, pos, pos_skip)
    h = jnp.concatenate([xi, x_skip], axis=1)
    h = jnp.zeros((_NPAD, 192), jnp.float32).at[:_NF].set(h)

    E = edge_index.shape[1]
    src = jnp.full((_E_PAD,), 0, jnp.int32).at[:E].set(edge_index[0])
    dst = jnp.full((_E_PAD,), _NF, jnp.int32).at[:E].set(edge_index[1])
    px = jnp.zeros((_NP2,), jnp.float32).at[:_NF].set(pos_skip[:, 0])
    py = jnp.zeros((_NP2,), jnp.float32).at[:_NF].set(pos_skip[:, 1])
    pz = jnp.zeros((_NP2,), jnp.float32).at[:_NF].set(pos_skip[:, 2])

    cnt = _sc_cnt(dst).reshape(32, _NP2).sum(axis=0)
    cnt8 = jnp.broadcast_to(cnt[:, None], (_NP2, 8))

    for (W, root, b) in ((W0, root0, b0), (W1, root1, b1), (W2, root2, b2)):
        ci = W.shape[1]
        Wr = W.transpose(1, 0, 2).reshape(ci, 125 * 64)
        table = _table_matmul(
            h[:, :ci].astype(jnp.bfloat16),
            Wr.astype(jnp.bfloat16)).reshape(_NPAD * 125 // 2, 128)
        aggp = _sc_layer(table, src, dst, px, py, pz)
        aggp = aggp.reshape(2, _NP2, 64)
        hn = _finalize(aggp[0], aggp[1], cnt8, h[:, :ci], root, b)
        h = hn.at[_NF:].set(0.0)

    return (h[:_NF], pos_skip, batch_skip)


# speedup vs baseline: 1.0331x; 1.0331x over previous
import dataclasses
import functools
import jax
import jax.numpy as jnp
from jax import lax
from jax.experimental import pallas as pl
from jax.experimental.pallas import tpu as pltpu
from jax.experimental.pallas import tpu_sc as plsc

_K = 5
_DIM = 3
_PATCH = 0.25
_KNN = 3

_BQ = 512
_NCP = 2560
_PADVAL = 1e4

_NF = 10000
_NP2 = 10240
_NPAD = 10240
_EPT = 5120
_E_PAD = 32 * _EPT
_CHUNK = 16
_NROW = _NP2 // 4


def _sc_compiler_params():
    cp = pltpu.CompilerParams()
    if "needs_layout_passes" in pltpu.CompilerParams.__dataclass_fields__:
        cp = dataclasses.replace(cp, needs_layout_passes=False)
    return cp


def _knn_body(q_ref, pT_ref, x_ref, o_ref):
    q = q_ref[...]
    pT = pT_ref[...]
    x = x_ref[...]
    qp = jnp.dot(q, pT, preferred_element_type=jnp.float32)
    pn = jnp.sum(pT * pT, axis=0, keepdims=True)
    qn = jnp.sum(q * q, axis=1, keepdims=True)
    d2 = qn + pn - 2.0 * qp
    colid = jax.lax.broadcasted_iota(jnp.int32, d2.shape, 1)
    mask = jnp.zeros(d2.shape, jnp.bool_)
    cur = d2
    for _ in range(_KNN):
        m = jnp.min(cur, axis=1, keepdims=True)
        is_min = cur == m
        first = jnp.min(jnp.where(is_min, colid, jnp.int32(2**30)),
                        axis=1, keepdims=True)
        sel = colid == first
        mask = mask | sel
        cur = jnp.where(sel, jnp.float32(1e30), cur)
    w = jnp.where(mask, 1.0 / jnp.maximum(d2, 1e-16), 0.0)
    s = jnp.sum(w, axis=1, keepdims=True)
    xi = jnp.dot(w, x, preferred_element_type=jnp.float32) / s
    o_ref[...] = xi


def _knn_interpolate_pallas(x, pos, pos_skip):
    Nc = pos.shape[0]
    Nf = pos_skip.shape[0]
    F = x.shape[1]
    nq_pad = ((Nf + _BQ - 1) // _BQ) * _BQ
    pT = jnp.full((8, _NCP), _PADVAL, jnp.float32)
    pT = pT.at[:3, :Nc].set(pos.T)
    pT = pT.at[3:, :Nc].set(0.0)
    xp = jnp.zeros((_NCP, F), jnp.float32).at[:Nc].set(x)
    qp = jnp.zeros((nq_pad, 8), jnp.float32).at[:Nf, :3].set(pos_skip)
    out = pl.pallas_call(
        _knn_body,
        grid=(nq_pad // _BQ,),
        in_specs=[
            pl.BlockSpec((_BQ, 8), lambda i: (i, 0)),
            pl.BlockSpec((8, _NCP), lambda i: (0, 0)),
            pl.BlockSpec((_NCP, F), lambda i: (0, 0)),
        ],
        out_specs=pl.BlockSpec((_BQ, F), lambda i: (i, 0)),
        out_shape=jax.ShapeDtypeStruct((nq_pad, F), jnp.float32),
    )(qp, pT, xp)
    return out[:Nf]


def _mm_body(h_ref, w_ref, o_ref):
    o_ref[...] = jnp.dot(h_ref[...], w_ref[...],
                         preferred_element_type=jnp.float32)


def _table_matmul(h, Wr):
    ci = h.shape[1]
    M = Wr.shape[1]
    return pl.pallas_call(
        _mm_body,
        grid=(_NPAD // _BQ,),
        in_specs=[
            pl.BlockSpec((_BQ, ci), lambda i: (i, 0)),
            pl.BlockSpec((ci, M), lambda i: (0, 0)),
        ],
        out_specs=pl.BlockSpec((_BQ, M), lambda i: (i, 0)),
        out_shape=jax.ShapeDtypeStruct((_NPAD, M), jnp.float32),
    )(h, Wr)


def _sc_layer(table, srcv_h, dstv_h, px, py, pz):
    mesh = plsc.VectorSubcoreMesh(core_axis_name="c", subcore_axis_name="s")

    @functools.partial(
        pl.kernel, mesh=mesh, compiler_params=_sc_compiler_params(),
        out_type=jax.ShapeDtypeStruct((2 * _NROW, 2, 128), jnp.float32),
        scratch_types=[
            pltpu.VMEM((_EPT,), jnp.int32),
            pltpu.VMEM((_EPT,), jnp.int32),
            pltpu.VMEM((_NP2,), jnp.float32),
            pltpu.VMEM((_NP2,), jnp.float32),
            pltpu.VMEM((_NP2,), jnp.float32),
            pltpu.VMEM((128,), jnp.int32),
            pltpu.VMEM((128,), jnp.int32),
            pltpu.VMEM((128, 128), jnp.float32),
            pltpu.VMEM((128, 128), jnp.float32),
            pltpu.VMEM((16, 2, 128), jnp.float32),
            pltpu.VMEM((16, 2, 128), jnp.float32),
            pltpu.VMEM((1, 16), jnp.int32),
            pltpu.VMEM((1, 16), jnp.int32),
            pltpu.SemaphoreType.DMA,
            pltpu.SemaphoreType.DMA,
            pltpu.SemaphoreType.DMA,
            pltpu.SemaphoreType.DMA,
            pltpu.VMEM_SHARED((_NROW, 2, 128), jnp.float32),
        ],
    )
    def k(table_h, src_h, dst_h, px_h, py_h, pz_h, out_h,
          srcv, dstv, pxv, pyv, pzv, qidxA, qidxB, gbufA, gbufB,
          sbufA, sbufB, dstrowA, dstrowB, semgA, semgB, semsA, semsB,
          agg_sh):
        cid = lax.axis_index("c")
        sid = lax.axis_index("s")
        wid = sid * 2 + cid

        zero16f = jnp.zeros((16,), jnp.float32)
        zero16i = jnp.zeros((16,), jnp.int32)

        @pl.loop(0, 16)
        def _(r):
            for t in range(2):
                for j in range(8):
                    sbufA.at[r, t][pl.ds(j * 16, 16)] = zero16f
                    sbufB.at[r, t][pl.ds(j * 16, 16)] = zero16f

        zbase = sid * (_NROW // 16)

        @pl.loop(0, _NROW // 16 // 16)
        def _(part):
            pltpu.sync_copy(sbufA, agg_sh.at[pl.ds(zbase + part * 16, 16)])
        plsc.subcore_barrier()

        base = wid * _EPT
        pltpu.sync_copy(src_h.at[pl.ds(base, _EPT)], srcv)
        pltpu.sync_copy(dst_h.at[pl.ds(base, _EPT)], dstv)
        pltpu.sync_copy(px_h, pxv)
        pltpu.sync_copy(py_h, pyv)
        pltpu.sync_copy(pz_h, pzv)

        def phase_p(c, qidx_q, gbuf_q, semg):
            off = c * _CHUNK
            s = srcv[pl.ds(off, 16)]
            d = dstv[pl.ds(off, 16)]

            def dim_corners(pv):
                gs = plsc.load_gather(pv, [s])
                gd = plsc.load_gather(pv, [d])
                v = jnp.clip((gd - gs) * 16.0 + 2.0, 0.0, 4.0)
                loi = v.astype(jnp.int32)
                fr = v - loi.astype(jnp.float32)
                return fr, loi, jnp.minimum(loi + 1, 4)

            fx, kx0, kx1 = dim_corners(pxv)
            fy, ky0, ky1 = dim_corners(pyv)
            fz, kz0, kz1 = dim_corners(pzv)
            gb = s * 125
            ax0 = gb + kx0
            ax1 = gb + kx1
            by0 = ky0 * 5
            by1 = ky1 * 5
            bz0 = kz0 * 25
            bz1 = kz1 * 25
            fx0 = 1.0 - fx
            fy0 = 1.0 - fy
            fz0 = 1.0 - fz
            wxy = (fx0 * fy0, fx0 * fy, fx * fy0, fx * fy)
            axy = (ax0 + by0, ax0 + by1, ax1 + by0, ax1 + by1)

            ws = []
            ps = []
            g = 0
            for t in range(4):
                for (fz_t, bz_t) in ((fz0, bz0), (fz, bz1)):
                    w8 = wxy[t] * fz_t
                    g8 = axy[t] + bz_t
                    qidx_q[pl.ds(g * 16, 16)] = g8 >> 1
                    ws.append(w8)
                    ps.append((g8 & 1) << 6)
                    g += 1

            pltpu.async_copy(table_h.at[qidx_q], gbuf_q, semg)
            return tuple(ws) + tuple(ps) + (d,)

        def phase_d(st, qidx_q, gbuf_q, sbuf_q, dstrow_q, semg, sems):
            ws = st[0:8]
            ps = st[8:16]
            d = st[16]
            pltpu.make_async_copy(table_h.at[qidx_q], gbuf_q, semg).wait()
            pltpu.make_async_copy(sbuf_q, agg_sh.at[dstrow_q.at[0]],
                                  sems).wait()
            dstrow_q.at[0][pl.ds(0, 16)] = d >> 2
            qv = (d & 3) << 6

            for e in range(16):
                accs = None
                for g in range(8):
                    w = ws[g][e]
                    p = ps[g][e]
                    row = gbuf_q.at[g * 16 + e]
                    vals = [row[pl.ds(p + j * 16, 16)] * w for j in range(4)]
                    if accs is None:
                        accs = vals
                    else:
                        accs = [a + v for a, v in zip(accs, vals)]
                qo = qv[e]
                for q in range(4):
                    t_s = q // 2
                    o_s = (q % 2) * 64
                    m = qo == q * 64
                    for j in range(4):
                        sbuf_q.at[e, t_s][pl.ds(o_s + j * 16, 16)] = (
                            jnp.where(m, accs[j], zero16f))

            pltpu.async_copy(sbuf_q, agg_sh.at[dstrow_q.at[0]], sems,
                             add=True)

        dstrowA.at[0][pl.ds(0, 16)] = zero16i
        dstrowB.at[0][pl.ds(0, 16)] = zero16i
        pltpu.async_copy(sbufA, agg_sh.at[dstrowA.at[0]], semsA, add=True)
        pltpu.async_copy(sbufB, agg_sh.at[dstrowB.at[0]], semsB, add=True)

        stA0 = phase_p(0, qidxA, gbufA, semgA)

        def pair_body(i, stA):
            c = 2 * i
            stB = phase_p(c + 1, qidxB, gbufB, semgB)
            phase_d(stA, qidxA, gbufA, sbufA, dstrowA, semgA, semsA)
            stA2 = phase_p(c + 2, qidxA, gbufA, semgA)
            phase_d(stB, qidxB, gbufB, sbufB, dstrowB, semgB, semsB)
            return stA2

        nchunk = _EPT // _CHUNK
        stA = lax.fori_loop(0, nchunk // 2 - 1, pair_body, stA0)

        stB = phase_p(nchunk - 1, qidxB, gbufB, semgB)
        phase_d(stA, qidxA, gbufA, sbufA, dstrowA, semgA, semsA)
        phase_d(stB, qidxB, gbufB, sbufB, dstrowB, semgB, semsB)

        pltpu.make_async_copy(sbufA, agg_sh.at[dstrowA.at[0]], semsA).wait()
        pltpu.make_async_copy(sbufB, agg_sh.at[dstrowB.at[0]], semsB).wait()

        plsc.subcore_barrier()

        obase = sid * (_NROW // 16)

        @pl.loop(0, _NROW // 16 // 16)
        def _(part):
            ooff = obase + part * 16
            pltpu.sync_copy(agg_sh.at[pl.ds(ooff, 16)], sbufA)
            pltpu.sync_copy(sbufA, out_h.at[pl.ds(cid * _NROW + ooff, 16)])

    return k(table, srcv_h, dstv_h, px, py, pz)


def _sc_cnt(dstv_h):
    mesh = plsc.VectorSubcoreMesh(core_axis_name="c", subcore_axis_name="s")

    @functools.partial(
        pl.kernel, mesh=mesh, compiler_params=_sc_compiler_params(),
        out_type=jax.ShapeDtypeStruct((32 * _NP2,), jnp.float32),
        scratch_types=[
            pltpu.VMEM((_EPT,), jnp.int32),
            pltpu.VMEM((_NP2,), jnp.float32),
        ],
    )
    def k(dst_h, out_h, dstv, hist):
        cid = lax.axis_index("c")
        sid = lax.axis_index("s")
        wid = sid * 2 + cid
        zero16 = jnp.zeros((16,), jnp.float32)
        one16 = jnp.full((16,), 1.0, jnp.float32)

        @pl.loop(0, _NP2, step=16)
        def _(i):
            hist[pl.ds(i, 16)] = zero16

        base = wid * _EPT
        pltpu.sync_copy(dst_h.at[pl.ds(base, _EPT)], dstv)

        @pl.loop(0, _EPT // 16)
        def _(c):
            d = dstv[pl.ds(c * 16, 16)]
            plsc.addupdate_scatter(hist, [d], one16)

        pltpu.sync_copy(hist, out_h.at[pl.ds(wid * _NP2, _NP2)])

    return k(dstv_h)


def _fin_body(a0_ref, a1_ref, c_ref, h_ref, root_ref, b_ref, o_ref):
    agg = a0_ref[...] + a1_ref[...]
    cnt = jnp.maximum(c_ref[:, 0:1], 1.0)
    hr = jnp.dot(h_ref[...], root_ref[...], preferred_element_type=jnp.float32)
    out = agg / cnt + hr + b_ref[0:1, :]
    out = jnp.where(out > 0, out, jnp.exp(jnp.minimum(out, 0.0)) - 1.0)
    row = (pl.program_id(0) * _BQ
           + jax.lax.broadcasted_iota(jnp.int32, out.shape, 0))
    o_ref[...] = jnp.where(row < _NF, out, 0.0)


def _finalize(agg0, agg1, cnt8, h, root, bias):
    ci = h.shape[1]
    bias8 = jnp.broadcast_to(bias[None, :], (8, 64))
    return pl.pallas_call(
        _fin_body,
        grid=(_NPAD // _BQ,),
        in_specs=[
            pl.BlockSpec((_BQ, 64), lambda i: (i, 0)),
            pl.BlockSpec((_BQ, 64), lambda i: (i, 0)),
            pl.BlockSpec((_BQ, 8), lambda i: (i, 0)),
            pl.BlockSpec((_BQ, ci), lambda i: (i, 0)),
            pl.BlockSpec((ci, 64), lambda i: (0, 0)),
            pl.BlockSpec((8, 64), lambda i: (0, 0)),
        ],
        out_specs=pl.BlockSpec((_BQ, 64), lambda i: (i, 0)),
        out_shape=jax.ShapeDtypeStruct((_NPAD, 64), jnp.float32),
    )(agg0, agg1, cnt8, h, root, bias8)


def kernel(x, pos, batch, x_skip, pos_skip, batch_skip, edge_index,
           W0, root0, b0, W1, root1, b1, W2, root2, b2):
    xi = _knn_interpolate_pallas(x, pos, pos_skip)
    h = jnp.concatenate([xi, x_skip], axis=1)
    h = jnp.zeros((_NPAD, 192), jnp.float32).at[:_NF].set(h)

    E = edge_index.shape[1]
    src = jnp.full((_E_PAD,), 0, jnp.int32).at[:E].set(edge_index[0])
    dst = jnp.full((_E_PAD,), _NF, jnp.int32).at[:E].set(edge_index[1])
    px = jnp.zeros((_NP2,), jnp.float32).at[:_NF].set(pos_skip[:, 0])
    py = jnp.zeros((_NP2,), jnp.float32).at[:_NF].set(pos_skip[:, 1])
    pz = jnp.zeros((_NP2,), jnp.float32).at[:_NF].set(pos_skip[:, 2])

    cnt = _sc_cnt(dst).reshape(32, _NP2).sum(axis=0)
    cnt8 = jnp.broadcast_to(cnt[:, None], (_NP2, 8))

    for (W, root, b) in ((W0, root0, b0), (W1, root1, b1), (W2, root2, b2)):
        ci = W.shape[1]
        Wr = W.transpose(1, 0, 2).reshape(ci, 125 * 64)
        table = _table_matmul(h[:, :ci], Wr).reshape(_NPAD * 125 // 2, 128)
        aggp = _sc_layer(table, src, dst, px, py, pz)
        aggp = aggp.reshape(2, _NP2, 64)
        h = _finalize(aggp[0], aggp[1], cnt8, h[:, :ci], root, b)

    return (h[:_NF], pos_skip, batch_skip)
